# baseline (speedup 1.0000x reference)
import jax
import jax.numpy as jnp
from jax import lax
from jax.experimental import pallas as pl
from jax.experimental.pallas import tpu as pltpu

_EMB = 1024
_DFF = 2048
_E = 8
_W_IMPORTANCE = 0.01
_TM = 1024
_TD = 1024


def _gate_kernel(lg_ref, prob_ref, ct_ref, imp_ref):
    t = pl.program_id(0)
    logits = lg_ref[...]
    lmax = jnp.max(logits, axis=1, keepdims=True)
    ex = jnp.exp(logits - lmax)
    p = ex / jnp.sum(ex, axis=1, keepdims=True)
    prob_ref[...] = p

    iota = lax.broadcasted_iota(jnp.int32, p.shape, 1)
    m1 = jnp.max(p, axis=1, keepdims=True)
    i1 = jnp.min(jnp.where(p == m1, iota, _E), axis=1, keepdims=True)
    oh1 = iota == i1
    pm = jnp.where(oh1, -jnp.inf, p)
    m2 = jnp.max(pm, axis=1, keepdims=True)
    i2 = jnp.min(jnp.where(pm == m2, iota, _E), axis=1, keepdims=True)
    oh2 = iota == i2
    e21 = jnp.exp(m2 - m1)
    w1 = 1.0 / (1.0 + e21)
    w2 = e21 / (1.0 + e21)
    ct_ref[...] = jnp.where(oh1, w1, 0.0) + jnp.where(oh2, w2, 0.0)

    @pl.when(t == 0)
    def _():
        imp_ref[...] = jnp.zeros_like(imp_ref)

    imp_ref[...] += jnp.sum(p, axis=0, keepdims=True)


def _moe_kernel(x_ref, w1_ref, b1_ref, w2_ref, b2_ref, c_ref, y_ref):
    e = pl.program_id(1)
    h = jnp.dot(x_ref[...], w1_ref[0], preferred_element_type=jnp.float32)
    h = jnp.maximum(h + b1_ref[0], 0.0).astype(jnp.bfloat16)
    part = jnp.dot(h, w2_ref[0], preferred_element_type=jnp.float32)
    cb = c_ref[0]

    @pl.when(e == 0)
    def _():
        y_ref[...] = jnp.zeros_like(y_ref)

    y_ref[...] += (part + b2_ref[0]) * cb


def kernel(x, q, Wq, bq, Wk, bk, Wv, bv, Wo, bo, gate_W, gate_b, W1, b1, W2, b2):
    x_shape = x.shape
    xf = x.reshape(-1, x_shape[-1])
    N, d = xf.shape
    T = N // _TM

    att = (q @ Wv + bv) @ Wo + bo
    gate_logits = att @ gate_W + gate_b

    gate_prob, c, imp = pl.pallas_call(
        _gate_kernel,
        grid=(T,),
        in_specs=[
            pl.BlockSpec((_TM, _E), lambda t: (t, 0)),
        ],
        out_specs=[
            pl.BlockSpec((_TM, _E), lambda t: (t, 0)),
            pl.BlockSpec((_TM, _E), lambda t: (t, 0)),
            pl.BlockSpec((1, _E), lambda t: (0, 0)),
        ],
        out_shape=[
            jax.ShapeDtypeStruct((N, _E), jnp.float32),
            jax.ShapeDtypeStruct((N, _E), jnp.float32),
            jax.ShapeDtypeStruct((1, _E), jnp.float32),
        ],
    )(gate_logits)

    cT = c.T.reshape(_E, N, 1)
    xb = xf.astype(jnp.bfloat16)
    w1b = W1.astype(jnp.bfloat16)
    w2b = W2.astype(jnp.bfloat16)
    b1r = b1.reshape(_E, 1, _DFF)
    b2r = b2.reshape(_E, 1, d)

    y = pl.pallas_call(
        _moe_kernel,
        grid=(N // _TD, _E),
        in_specs=[
            pl.BlockSpec((_TD, d), lambda i, e: (i, 0)),
            pl.BlockSpec((1, d, _DFF), lambda i, e: (e, 0, 0)),
            pl.BlockSpec((1, 1, _DFF), lambda i, e: (e, 0, 0)),
            pl.BlockSpec((1, _DFF, d), lambda i, e: (e, 0, 0)),
            pl.BlockSpec((1, 1, d), lambda i, e: (e, 0, 0)),
            pl.BlockSpec((1, _TD, 1), lambda i, e: (e, i, 0)),
        ],
        out_specs=pl.BlockSpec((_TD, d), lambda i, e: (i, 0)),
        out_shape=jax.ShapeDtypeStruct((N, d), jnp.float32),
    )(xb, w1b, b1r, w2b, b2r, cT)

    importance = imp[0]
    importance_loss = _W_IMPORTANCE * (
        jnp.std(importance, ddof=1) / jnp.mean(importance)) ** 2
    return y.reshape(x_shape), gate_prob, importance_loss
